# baseline (device time: 113771 ns/iter reference)
import jax
import jax.numpy as jnp
from jax import lax
from jax.experimental import pallas as pl
from jax.experimental.pallas import tpu as pltpu

N_DEV = 4
N_LOCAL_EXPERTS = 8
CAP = 640


def kernel(x, router_W, route_idx, expert_W, shared_W):
    n_tok, d_model = x.shape
    d_out = expert_W.shape[2]
    my = lax.axis_index("i")

    scores = x @ router_W
    probs = jax.nn.softmax(scores, axis=-1)
    gate = jnp.take_along_axis(probs, route_idx, axis=1)
    chip_of = route_idx[:, 0] // N_LOCAL_EXPERTS
    row_ids = jnp.stack(
        [jnp.argsort(chip_of != q, stable=True)[:CAP] for q in range(N_DEV)]
    ).astype(jnp.int32)
    mine = row_ids[my]
    xg = jnp.take(x, mine, axis=0) * jnp.take(gate, mine, axis=0)
    eg = jnp.take(route_idx, mine, axis=0)

    def body(xg_ref, eg_ref, x_ref, ew_ref, sw_ref, rid_ref, out_ref,
             acc_ref, sendbuf_ref, comm_ref, ew_vmem,
             ew_sems, send_sems, recv_sems):
        my_pos = lax.axis_index("i")

        def fetch(k, slot):
            pltpu.make_async_copy(
                ew_ref.at[k], ew_vmem.at[slot], ew_sems.at[slot]).start()

        fetch(0, 0)

        barrier_sem = pltpu.get_barrier_semaphore()
        for d in range(1, N_DEV):
            pl.semaphore_signal(
                barrier_sem, inc=1,
                device_id=(lax.rem(my_pos + d, N_DEV),),
                device_id_type=pl.DeviceIdType.MESH,
            )
        pl.semaphore_wait(barrier_sem, N_DEV - 1)

        eg = eg_ref[:, :]
        xg = xg_ref[:, :]
        for k in range(N_LOCAL_EXPERTS):
            slot = k % 2
            if k + 1 < N_LOCAL_EXPERTS:
                fetch(k + 1, (k + 1) % 2)
            pltpu.make_async_copy(
                ew_ref.at[k], ew_vmem.at[slot], ew_sems.at[slot]).wait()
            w = (eg == my_pos * N_LOCAL_EXPERTS + k).astype(jnp.float32)
            contrib = jnp.dot(xg * w, ew_vmem[slot],
                              preferred_element_type=jnp.float32)
            if k == 0:
                acc_ref[:, :] = contrib
            else:
                acc_ref[:, :] = acc_ref[:, :] + contrib

        sendbuf_ref[:, :] = acc_ref[:, :].astype(jnp.bfloat16)
        rdmas = []
        for d in range(1, N_DEV):
            rdma = pltpu.make_async_remote_copy(
                src_ref=sendbuf_ref,
                dst_ref=comm_ref.at[N_DEV - 1 - d],
                send_sem=send_sems.at[d - 1],
                recv_sem=recv_sems.at[N_DEV - 1 - d],
                device_id=(lax.rem(my_pos + d, N_DEV),),
                device_id_type=pl.DeviceIdType.MESH,
            )
            rdma.start()
            rdmas.append(rdma)

        out_ref[:, :] = jnp.dot(x_ref[:, :], sw_ref[:, :],
                                preferred_element_type=jnp.float32)

        def scatter_mat(q, dtype):
            ids = rid_ref[pl.ds(q, 1), :]
            i = lax.broadcasted_iota(jnp.int32, (n_tok, CAP), 0)
            return (i == ids).astype(dtype)

        out_ref[:, :] = out_ref[:, :] + jnp.dot(
            scatter_mat(my_pos, jnp.float32), acc_ref[:, :],
            preferred_element_type=jnp.float32)

        for j in range(N_DEV - 1):
            rdmas[N_DEV - 2 - j].wait_recv()
            sender = lax.rem(my_pos + 1 + j, N_DEV)
            out_ref[:, :] = out_ref[:, :] + jnp.dot(
                scatter_mat(sender, jnp.bfloat16), comm_ref[j],
                preferred_element_type=jnp.float32)

        for rdma in rdmas:
            rdma.wait_send()

    out = pl.pallas_call(
        body,
        out_shape=jax.ShapeDtypeStruct((n_tok, d_out), jnp.float32),
        in_specs=[
            pl.BlockSpec(memory_space=pltpu.VMEM),
            pl.BlockSpec(memory_space=pltpu.VMEM),
            pl.BlockSpec(memory_space=pltpu.VMEM),
            pl.BlockSpec(memory_space=pl.ANY),
            pl.BlockSpec(memory_space=pltpu.VMEM),
            pl.BlockSpec(memory_space=pltpu.VMEM),
        ],
        out_specs=pl.BlockSpec(memory_space=pltpu.VMEM),
        scratch_shapes=[
            pltpu.VMEM((CAP, d_out), jnp.float32),
            pltpu.VMEM((CAP, d_out), jnp.bfloat16),
            pltpu.VMEM((N_DEV - 1, CAP, d_out), jnp.bfloat16),
            pltpu.VMEM((2, d_model, d_out), jnp.float32),
            pltpu.SemaphoreType.DMA((2,)),
            pltpu.SemaphoreType.DMA((N_DEV - 1,)),
            pltpu.SemaphoreType.DMA((N_DEV - 1,)),
        ],
        compiler_params=pltpu.CompilerParams(
            collective_id=0, vmem_limit_bytes=38 * 1024 * 1024),
    )(xg, eg, x, expert_W, shared_W, row_ids)
    return out


# device time: 87275 ns/iter; 1.3036x vs baseline; 1.3036x over previous
import jax
import jax.numpy as jnp
from jax import lax
from jax.experimental import pallas as pl
from jax.experimental.pallas import tpu as pltpu

N_DEV = 4
N_LOCAL_EXPERTS = 8
CAP = 640
CSUM_BLK = 512


def kernel(x, router_W, route_idx, expert_W, shared_W):
    n_tok, d_model = x.shape
    n_exp = router_W.shape[1]
    d_out = expert_W.shape[2]
    route_idx_t = route_idx.T

    def body(x_ref, rw_ref, idx_ref, idxt_ref, ew_ref, sw_ref, out_ref,
             acc_ref, sendbuf_ref, comm_ref, ew_vmem,
             ew_sems, send_sems, recv_sems):
        my_pos = lax.axis_index("i")

        def fetch(k, slot):
            pltpu.make_async_copy(
                ew_ref.at[k], ew_vmem.at[slot], ew_sems.at[slot]).start()

        fetch(0, 0)

        barrier_sem = pltpu.get_barrier_semaphore()
        for d in range(1, N_DEV):
            pl.semaphore_signal(
                barrier_sem, inc=1,
                device_id=(lax.rem(my_pos + d, N_DEV),),
                device_id_type=pl.DeviceIdType.MESH,
            )
        pl.semaphore_wait(barrier_sem, N_DEV - 1)

        ii = lax.broadcasted_iota(jnp.int32, (CSUM_BLK, CSUM_BLK), 0)
        jj = lax.broadcasted_iota(jnp.int32, (CSUM_BLK, CSUM_BLK), 1)
        tri_u = (ii <= jj).astype(jnp.float32)
        tri_l = (ii >= jj).astype(jnp.float32)

        chip_of_t = idxt_ref[:, :] // N_LOCAL_EXPERTS
        mine_t = (chip_of_t == my_pos).astype(jnp.float32)
        slot_parts = []
        carry = 0.0
        for c in range(n_tok // CSUM_BLK):
            blk = mine_t[:, c * CSUM_BLK:(c + 1) * CSUM_BLK]
            slot_parts.append(
                jnp.dot(blk, tri_u, preferred_element_type=jnp.float32)
                + carry)
            carry = carry + jnp.sum(blk)
        slot_t = jnp.concatenate(slot_parts, axis=1) - 1.0

        s_iota = lax.broadcasted_iota(jnp.int32, (CAP, n_tok), 0)
        gather = jnp.where(
            (s_iota == slot_t.astype(jnp.int32)) & (mine_t == 1.0),
            1.0, 0.0)

        scores = jnp.dot(x_ref[:, :], rw_ref[:, :],
                         preferred_element_type=jnp.float32)
        mx = jnp.max(scores, axis=-1, keepdims=True)
        ex = jnp.exp(scores - mx)
        probs = ex / jnp.sum(ex, axis=-1, keepdims=True)
        eidx = idx_ref[:, :]
        lane = lax.broadcasted_iota(jnp.int32, (n_tok, n_exp), 1)
        gate = jnp.sum(jnp.where(lane == eidx, probs, 0.0), axis=-1,
                       keepdims=True)

        gate_g = jnp.dot(gather, gate, preferred_element_type=jnp.float32)
        eg_f = jnp.dot(gather, eidx.astype(jnp.float32),
                       preferred_element_type=jnp.float32)
        xg = jnp.dot(gather, x_ref[:, :],
                     preferred_element_type=jnp.float32) * gate_g

        for k in range(N_LOCAL_EXPERTS):
            slot = k % 2
            if k + 1 < N_LOCAL_EXPERTS:
                fetch(k + 1, (k + 1) % 2)
            pltpu.make_async_copy(
                ew_ref.at[k], ew_vmem.at[slot], ew_sems.at[slot]).wait()
            w = (eg_f == (my_pos * N_LOCAL_EXPERTS + k).astype(jnp.float32)
                 ).astype(jnp.float32)
            contrib = jnp.dot(xg * w, ew_vmem[slot],
                              preferred_element_type=jnp.float32)
            if k == 0:
                acc_ref[:, :] = contrib
            else:
                acc_ref[:, :] = acc_ref[:, :] + contrib

        sendbuf_ref[:, :] = acc_ref[:, :].astype(jnp.bfloat16)
        rdmas = []
        for d in range(1, N_DEV):
            rdma = pltpu.make_async_remote_copy(
                src_ref=sendbuf_ref,
                dst_ref=comm_ref.at[N_DEV - 1 - d],
                send_sem=send_sems.at[d - 1],
                recv_sem=recv_sems.at[N_DEV - 1 - d],
                device_id=(lax.rem(my_pos + d, N_DEV),),
                device_id_type=pl.DeviceIdType.MESH,
            )
            rdma.start()
            rdmas.append(rdma)

        chip_of = eidx // N_LOCAL_EXPERTS
        chips = lax.broadcasted_iota(jnp.int32, (n_tok, N_DEV), 1)
        memb = (chip_of == chips).astype(jnp.float32)
        cs_parts = []
        carry_c = jnp.zeros((1, N_DEV), jnp.float32)
        for c in range(n_tok // CSUM_BLK):
            blk = memb[c * CSUM_BLK:(c + 1) * CSUM_BLK, :]
            cs_parts.append(
                jnp.dot(tri_l, blk, preferred_element_type=jnp.float32)
                + carry_c)
            carry_c = carry_c + jnp.sum(blk, axis=0, keepdims=True)
        cs = jnp.concatenate(cs_parts, axis=0) - 1.0

        def scatter_mat(q, dtype):
            onehot_q = (lax.broadcasted_iota(jnp.int32, (1, N_DEV), 1)
                        == q).astype(jnp.float32)
            m_q = jnp.sum(memb * onehot_q, axis=1, keepdims=True)
            slot_q = jnp.sum(cs * onehot_q, axis=1, keepdims=True)
            c_iota = lax.broadcasted_iota(jnp.int32, (n_tok, CAP), 1)
            return jnp.where(
                (c_iota == slot_q.astype(jnp.int32)) & (m_q == 1.0),
                1.0, 0.0).astype(dtype)

        out_ref[:, :] = jnp.dot(x_ref[:, :], sw_ref[:, :],
                                preferred_element_type=jnp.float32)
        out_ref[:, :] = out_ref[:, :] + jnp.dot(
            scatter_mat(my_pos, jnp.float32), acc_ref[:, :],
            preferred_element_type=jnp.float32)

        for j in range(N_DEV - 1):
            rdmas[N_DEV - 2 - j].wait_recv()
            sender = lax.rem(my_pos + 1 + j, N_DEV)
            out_ref[:, :] = out_ref[:, :] + jnp.dot(
                scatter_mat(sender, jnp.bfloat16), comm_ref[j],
                preferred_element_type=jnp.float32)

        for rdma in rdmas:
            rdma.wait_send()

    out = pl.pallas_call(
        body,
        out_shape=jax.ShapeDtypeStruct((n_tok, d_out), jnp.float32),
        in_specs=[
            pl.BlockSpec(memory_space=pltpu.VMEM),
            pl.BlockSpec(memory_space=pltpu.VMEM),
            pl.BlockSpec(memory_space=pltpu.VMEM),
            pl.BlockSpec(memory_space=pltpu.VMEM),
            pl.BlockSpec(memory_space=pl.ANY),
            pl.BlockSpec(memory_space=pltpu.VMEM),
        ],
        out_specs=pl.BlockSpec(memory_space=pltpu.VMEM),
        scratch_shapes=[
            pltpu.VMEM((CAP, d_out), jnp.float32),
            pltpu.VMEM((CAP, d_out), jnp.bfloat16),
            pltpu.VMEM((N_DEV - 1, CAP, d_out), jnp.bfloat16),
            pltpu.VMEM((2, d_model, d_out), jnp.float32),
            pltpu.SemaphoreType.DMA((2,)),
            pltpu.SemaphoreType.DMA((N_DEV - 1,)),
            pltpu.SemaphoreType.DMA((N_DEV - 1,)),
        ],
        compiler_params=pltpu.CompilerParams(
            collective_id=0, vmem_limit_bytes=44 * 1024 * 1024),
    )(x, router_W, route_idx, route_idx_t, expert_W, shared_W)
    return out


# device time: 85623 ns/iter; 1.3287x vs baseline; 1.0193x over previous
import jax
import jax.numpy as jnp
from jax import lax
from jax.experimental import pallas as pl
from jax.experimental.pallas import tpu as pltpu

N_DEV = 4
N_LOCAL_EXPERTS = 8
CAP = 640
CSUM_BLK = 512


def kernel(x, router_W, route_idx, expert_W, shared_W):
    n_tok, d_model = x.shape
    n_exp = router_W.shape[1]
    d_out = expert_W.shape[2]
    route_idx_t = route_idx.T

    def body(x_ref, rw_ref, idx_ref, idxt_ref, ew_ref, sw_ref, out_ref,
             acc_ref, sendbuf_ref, comm_ref, ew_vmem,
             ew_sems, send_sems, recv_sems):
        my_pos = lax.axis_index("i")

        def fetch(k, slot):
            pltpu.make_async_copy(
                ew_ref.at[k], ew_vmem.at[slot], ew_sems.at[slot]).start()

        fetch(0, 0)

        barrier_sem = pltpu.get_barrier_semaphore()
        for d in range(1, N_DEV):
            pl.semaphore_signal(
                barrier_sem, inc=1,
                device_id=(lax.rem(my_pos + d, N_DEV),),
                device_id_type=pl.DeviceIdType.MESH,
            )

        ii = lax.broadcasted_iota(jnp.int32, (CSUM_BLK, CSUM_BLK), 0)
        jj = lax.broadcasted_iota(jnp.int32, (CSUM_BLK, CSUM_BLK), 1)
        tri_u = (ii <= jj).astype(jnp.float32)
        tri_l = (ii >= jj).astype(jnp.float32)

        chip_of_t = idxt_ref[:, :] // N_LOCAL_EXPERTS
        mine_t = (chip_of_t == my_pos).astype(jnp.float32)
        slot_parts = []
        carry = 0.0
        for c in range(n_tok // CSUM_BLK):
            blk = mine_t[:, c * CSUM_BLK:(c + 1) * CSUM_BLK]
            slot_parts.append(
                jnp.dot(blk, tri_u, preferred_element_type=jnp.float32)
                + carry)
            carry = carry + jnp.sum(blk)
        slot_t = jnp.concatenate(slot_parts, axis=1) - 1.0

        s_iota = lax.broadcasted_iota(jnp.int32, (CAP, n_tok), 0)
        gather = jnp.where(
            (s_iota == slot_t.astype(jnp.int32)) & (mine_t == 1.0),
            1.0, 0.0)

        eidx = idx_ref[:, :]
        eg_f = jnp.dot(gather, eidx.astype(jnp.float32),
                       preferred_element_type=jnp.float32)
        xg = jnp.dot(gather, x_ref[:, :],
                     preferred_element_type=jnp.float32)

        for k in range(N_LOCAL_EXPERTS):
            slot = k % 2
            if k + 1 < N_LOCAL_EXPERTS:
                fetch(k + 1, (k + 1) % 2)
            pltpu.make_async_copy(
                ew_ref.at[k], ew_vmem.at[slot], ew_sems.at[slot]).wait()
            w = (eg_f == (my_pos * N_LOCAL_EXPERTS + k).astype(jnp.float32)
                 ).astype(jnp.float32)
            contrib = jnp.dot(xg * w, ew_vmem[slot],
                              preferred_element_type=jnp.float32)
            if k == 0:
                acc_ref[:, :] = contrib
            else:
                acc_ref[:, :] = acc_ref[:, :] + contrib

        sendbuf_ref[:, :] = acc_ref[:, :].astype(jnp.bfloat16)
        pl.semaphore_wait(barrier_sem, N_DEV - 1)
        rdmas = []
        for d in range(1, N_DEV):
            rdma = pltpu.make_async_remote_copy(
                src_ref=sendbuf_ref,
                dst_ref=comm_ref.at[N_DEV - 1 - d],
                send_sem=send_sems.at[d - 1],
                recv_sem=recv_sems.at[N_DEV - 1 - d],
                device_id=(lax.rem(my_pos + d, N_DEV),),
                device_id_type=pl.DeviceIdType.MESH,
            )
            rdma.start()
            rdmas.append(rdma)

        scores = jnp.dot(x_ref[:, :], rw_ref[:, :],
                         preferred_element_type=jnp.float32)
        mx = jnp.max(scores, axis=-1, keepdims=True)
        ex = jnp.exp(scores - mx)
        probs = ex / jnp.sum(ex, axis=-1, keepdims=True)
        lane = lax.broadcasted_iota(jnp.int32, (n_tok, n_exp), 1)
        gate = jnp.sum(jnp.where(lane == eidx, probs, 0.0), axis=-1,
                       keepdims=True)

        chip_of = eidx // N_LOCAL_EXPERTS
        chips = lax.broadcasted_iota(jnp.int32, (n_tok, N_DEV), 1)
        memb = (chip_of == chips).astype(jnp.float32)
        cs_parts = []
        carry_c = jnp.zeros((1, N_DEV), jnp.float32)
        for c in range(n_tok // CSUM_BLK):
            blk = memb[c * CSUM_BLK:(c + 1) * CSUM_BLK, :]
            cs_parts.append(
                jnp.dot(tri_l, blk, preferred_element_type=jnp.float32)
                + carry_c)
            carry_c = carry_c + jnp.sum(blk, axis=0, keepdims=True)
        cs = jnp.concatenate(cs_parts, axis=0) - 1.0

        def scatter_mat(q, dtype):
            onehot_q = (lax.broadcasted_iota(jnp.int32, (1, N_DEV), 1)
                        == q).astype(jnp.float32)
            m_q = jnp.sum(memb * onehot_q, axis=1, keepdims=True)
            slot_q = jnp.sum(cs * onehot_q, axis=1, keepdims=True)
            c_iota = lax.broadcasted_iota(jnp.int32, (n_tok, CAP), 1)
            return jnp.where(
                (c_iota == slot_q.astype(jnp.int32)) & (m_q == 1.0),
                gate, 0.0).astype(dtype)

        out_ref[:, :] = jnp.dot(x_ref[:, :], sw_ref[:, :],
                                preferred_element_type=jnp.float32)
        out_ref[:, :] = out_ref[:, :] + jnp.dot(
            scatter_mat(my_pos, jnp.float32), acc_ref[:, :],
            preferred_element_type=jnp.float32)

        for j in range(N_DEV - 1):
            rdmas[N_DEV - 2 - j].wait_recv()
            sender = lax.rem(my_pos + 1 + j, N_DEV)
            out_ref[:, :] = out_ref[:, :] + jnp.dot(
                scatter_mat(sender, jnp.bfloat16), comm_ref[j],
                preferred_element_type=jnp.float32)

        for rdma in rdmas:
            rdma.wait_send()

    out = pl.pallas_call(
        body,
        out_shape=jax.ShapeDtypeStruct((n_tok, d_out), jnp.float32),
        in_specs=[
            pl.BlockSpec(memory_space=pltpu.VMEM),
            pl.BlockSpec(memory_space=pltpu.VMEM),
            pl.BlockSpec(memory_space=pltpu.VMEM),
            pl.BlockSpec(memory_space=pltpu.VMEM),
            pl.BlockSpec(memory_space=pl.ANY),
            pl.BlockSpec(memory_space=pltpu.VMEM),
        ],
        out_specs=pl.BlockSpec(memory_space=pltpu.VMEM),
        scratch_shapes=[
            pltpu.VMEM((CAP, d_out), jnp.float32),
            pltpu.VMEM((CAP, d_out), jnp.bfloat16),
            pltpu.VMEM((N_DEV - 1, CAP, d_out), jnp.bfloat16),
            pltpu.VMEM((2, d_model, d_out), jnp.float32),
            pltpu.SemaphoreType.DMA((2,)),
            pltpu.SemaphoreType.DMA((N_DEV - 1,)),
            pltpu.SemaphoreType.DMA((N_DEV - 1,)),
        ],
        compiler_params=pltpu.CompilerParams(
            collective_id=0, vmem_limit_bytes=44 * 1024 * 1024),
    )(x, router_W, route_idx, route_idx_t, expert_W, shared_W)
    return out
